# baseline (speedup 1.0000x reference)
import functools

import jax
import jax.numpy as jnp
from jax import lax
from jax.experimental import pallas as pl
from jax.experimental.pallas import tpu as pltpu
from jax.experimental.pallas import tpu_sc as plsc

_UNITS = 26
_LOWER = -17.0
_UPPER = 8.0
_BW = 0.05
_NUM_BUCKETS = int((_UPPER - _LOWER) / _BW) + 1
_RESIDUE = _LOWER - _BW
_BATCH = 4096

_KPAD = 512
_NW = 32
_ELEMS = _BATCH * _UNITS
_EPW = _ELEMS // _NW
_VECS = _EPW // 16


def _table_kernel(w_ref, b_ref, t1_ref, t2_ref):
    w = jnp.maximum(w_ref[...], jnp.float32(0.0))
    r = lax.broadcasted_iota(jnp.int32, (_KPAD, _KPAD), 0)
    c = lax.broadcasted_iota(jnp.int32, (_KPAD, _KPAD), 1)
    tri = jnp.where(r < c, jnp.float32(_BW), jnp.float32(0.0))
    t1_ref[...] = (
        jnp.dot(w, tri, preferred_element_type=jnp.float32,
                precision=lax.Precision.HIGHEST)
        + (jnp.float32(_RESIDUE) + b_ref[...])
    )
    t2_ref[...] = w


def _build_tables(weights, bias):
    wp = jnp.pad(weights, ((0, 0), (0, _KPAD - _NUM_BUCKETS)))
    t1, t2 = pl.pallas_call(
        _table_kernel,
        out_shape=[
            jax.ShapeDtypeStruct((_UNITS, _KPAD), jnp.float32),
            jax.ShapeDtypeStruct((_UNITS, _KPAD), jnp.float32),
        ],
    )(wp, bias.reshape(_UNITS, 1))
    return t1.reshape(-1), t2.reshape(-1)


def _sc_body(x_hbm, fb_hbm, t1_hbm, t2_hbm, out_hbm,
             x_v, fb_v, out_v, t1_v, t2_v, sem):
    wid = lax.axis_index("s") * 2 + lax.axis_index("c")
    base = wid * _EPW
    c1 = pltpu.async_copy(x_hbm.at[pl.ds(base, _EPW)], x_v, sem)
    c2 = pltpu.async_copy(fb_hbm, fb_v, sem)
    c3 = pltpu.async_copy(t1_hbm, t1_v, sem)
    c4 = pltpu.async_copy(t2_hbm, t2_v, sem)
    c1.wait()
    c2.wait()
    c3.wait()
    c4.wait()

    def body(i, carry):
        off = i * 16
        xv = x_v[pl.ds(off, 16)]
        fb = fb_v[pl.ds(off, 16)]
        xc = jnp.clip(xv, jnp.float32(_LOWER + 1e-9), jnp.float32(_UPPER - 1e-9))
        t = (xc - jnp.float32(_LOWER) + jnp.float32(_BW)) * jnp.float32(1.0 / _BW)
        idx = jnp.clip(t.astype(jnp.int32), 0, _NUM_BUCKETS - 1)
        delta = (
            xc - jnp.float32(_LOWER) + jnp.float32(_BW)
            - idx.astype(jnp.float32) * jnp.float32(_BW)
        )
        fidx = fb + idx
        g1 = plsc.load_gather(t1_v, [fidx])
        g2 = plsc.load_gather(t2_v, [fidx])
        z = g1 + delta * g2
        out_v[pl.ds(off, 16)] = jnp.float32(1.0) / (jnp.float32(1.0) + jnp.exp(-z))
        return carry

    lax.fori_loop(0, _VECS, body, None, unroll=8)
    pltpu.sync_copy(out_v, out_hbm.at[pl.ds(base, _EPW)])


def kernel(x, weights, bias):
    t1, t2 = _build_tables(weights, bias)
    fb = (jnp.arange(_EPW, dtype=jnp.int32) % _UNITS) * _KPAD
    mesh = plsc.VectorSubcoreMesh(core_axis_name="c", subcore_axis_name="s")
    run = functools.partial(
        pl.kernel,
        mesh=mesh,
        out_type=jax.ShapeDtypeStruct((_ELEMS,), jnp.float32),
        scratch_types=[
            pltpu.VMEM((_EPW,), jnp.float32),
            pltpu.VMEM((_EPW,), jnp.int32),
            pltpu.VMEM((_EPW,), jnp.float32),
            pltpu.VMEM((_UNITS * _KPAD,), jnp.float32),
            pltpu.VMEM((_UNITS * _KPAD,), jnp.float32),
            pltpu.SemaphoreType.DMA,
        ],
        compiler_params=pltpu.CompilerParams(needs_layout_passes=False),
    )(_sc_body)
    out = run(x.reshape(-1), fb, t1, t2)
    return out.reshape(_BATCH, _UNITS)


# speedup vs baseline: 1.1825x; 1.1825x over previous
import functools

import jax
import jax.numpy as jnp
from jax import lax
from jax.experimental import pallas as pl
from jax.experimental.pallas import tpu as pltpu
from jax.experimental.pallas import tpu_sc as plsc

_UNITS = 26
_LOWER = -17.0
_UPPER = 8.0
_BW = 0.05
_NUM_BUCKETS = int((_UPPER - _LOWER) / _BW) + 1
_RESIDUE = _LOWER - _BW
_BATCH = 4096

_KPAD = 512
_NW = 32
_ELEMS = _BATCH * _UNITS
_EPW = _ELEMS // _NW
_VECS = _EPW // 16


def _table_kernel(w_ref, b_ref, t1_ref, t2_ref):
    w = jnp.maximum(w_ref[...], jnp.float32(0.0))
    r = lax.broadcasted_iota(jnp.int32, (_KPAD, _KPAD), 0)
    c = lax.broadcasted_iota(jnp.int32, (_KPAD, _KPAD), 1)
    tri = jnp.where(r < c, jnp.float32(_BW), jnp.float32(0.0))
    t1_ref[...] = (
        jnp.dot(w, tri, preferred_element_type=jnp.float32,
                precision=lax.Precision.HIGHEST)
        + (jnp.float32(_RESIDUE) + b_ref[...])
    )
    t2_ref[...] = w


def _build_tables(weights, bias):
    wp = jnp.pad(weights, ((0, 0), (0, _KPAD - _NUM_BUCKETS)))
    t1, t2 = pl.pallas_call(
        _table_kernel,
        out_shape=[
            jax.ShapeDtypeStruct((_UNITS, _KPAD), jnp.float32),
            jax.ShapeDtypeStruct((_UNITS, _KPAD), jnp.float32),
        ],
    )(wp, bias.reshape(_UNITS, 1))
    return t1.reshape(-1), t2.reshape(-1)


def _sc_body(x_hbm, t1_hbm, t2_hbm, out_hbm,
             x_v, out_v, t1_v, t2_v, sem):
    wid = lax.axis_index("s") * 2 + lax.axis_index("c")
    base = wid * _EPW
    c1 = pltpu.async_copy(x_hbm.at[pl.ds(base, _EPW)], x_v, sem)
    c2 = pltpu.async_copy(t1_hbm, t1_v, sem)
    c3 = pltpu.async_copy(t2_hbm, t2_v, sem)
    c1.wait()
    c2.wait()
    c3.wait()

    lane = lax.iota(jnp.int32, 16)

    @plsc.parallel_loop(0, _EPW, step=16, unroll=4)
    def _loop(off):
        xv = x_v[pl.ds(off, 16)]
        xc = jnp.clip(xv, jnp.float32(_LOWER + 1e-9), jnp.float32(_UPPER - 1e-9))
        t = (xc - jnp.float32(_LOWER) + jnp.float32(_BW)) * jnp.float32(1.0 / _BW)
        idx = jnp.clip(t.astype(jnp.int32), 0, _NUM_BUCKETS - 1)
        delta = (
            xc - jnp.float32(_LOWER) + jnp.float32(_BW)
            - idx.astype(jnp.float32) * jnp.float32(_BW)
        )
        u = jnp.remainder(base + off + lane, jnp.int32(_UNITS))
        fidx = u * _KPAD + idx
        g1 = plsc.load_gather(t1_v, [fidx])
        g2 = plsc.load_gather(t2_v, [fidx])
        z = g1 + delta * g2
        out_v[pl.ds(off, 16)] = jnp.float32(1.0) / (jnp.float32(1.0) + jnp.exp(-z))

    pltpu.sync_copy(out_v, out_hbm.at[pl.ds(base, _EPW)])


def kernel(x, weights, bias):
    t1, t2 = _build_tables(weights, bias)
    mesh = plsc.VectorSubcoreMesh(core_axis_name="c", subcore_axis_name="s")
    run = functools.partial(
        pl.kernel,
        mesh=mesh,
        out_type=jax.ShapeDtypeStruct((_ELEMS,), jnp.float32),
        scratch_types=[
            pltpu.VMEM((_EPW,), jnp.float32),
            pltpu.VMEM((_EPW,), jnp.float32),
            pltpu.VMEM((_UNITS * _KPAD,), jnp.float32),
            pltpu.VMEM((_UNITS * _KPAD,), jnp.float32),
            pltpu.SemaphoreType.DMA,
        ],
        compiler_params=pltpu.CompilerParams(needs_layout_passes=False),
    )(_sc_body)
    out = run(x.reshape(-1), t1, t2)
    return out.reshape(_BATCH, _UNITS)


# speedup vs baseline: 1.7464x; 1.4769x over previous
import functools

import jax
import jax.numpy as jnp
from jax import lax
from jax.experimental import pallas as pl
from jax.experimental.pallas import tpu as pltpu
from jax.experimental.pallas import tpu_sc as plsc

_UNITS = 26
_LOWER = -17.0
_UPPER = 8.0
_BW = 0.05
_NUM_BUCKETS = int((_UPPER - _LOWER) / _BW) + 1
_RESIDUE = _LOWER - _BW
_BATCH = 4096

_KPAD = 512
_CHUNKS = _KPAD // 16


def _sc_body(x_hbm, w_hbm, b_hbm, out_hbm,
             x_v, out_v, w_v, b_v, t1_v, t2_v, sem):
    u = lax.axis_index("s") * 2 + lax.axis_index("c")

    @pl.when(u < _UNITS)
    def _():
        base = u * _BATCH
        c1 = pltpu.async_copy(x_hbm.at[pl.ds(base, _BATCH)], x_v, sem)
        c2 = pltpu.async_copy(w_hbm.at[pl.ds(u * _KPAD, _KPAD)], w_v, sem)
        c3 = pltpu.async_copy(b_hbm, b_v, sem)
        c2.wait()
        c3.wait()

        bias_u = plsc.load_gather(b_v, [u + jnp.zeros((16,), jnp.int32)])

        def chunk(c, carry):
            v = jnp.maximum(w_v[pl.ds(c * 16, 16)], jnp.float32(0.0))
            incl = plsc.cumsum(v)
            t1_v[pl.ds(c * 16, 16)] = (
                (incl - v + carry) * jnp.float32(_BW)
                + jnp.float32(_RESIDUE) + bias_u
            )
            t2_v[pl.ds(c * 16, 16)] = v
            return carry + jnp.sum(v)

        lax.fori_loop(0, _CHUNKS, chunk, jnp.zeros((16,), jnp.float32))
        c1.wait()

        @plsc.parallel_loop(0, _BATCH, step=16, unroll=4)
        def _loop(off):
            xv = x_v[pl.ds(off, 16)]
            xc = jnp.clip(xv, jnp.float32(_LOWER + 1e-9), jnp.float32(_UPPER - 1e-9))
            t = (xc - jnp.float32(_LOWER) + jnp.float32(_BW)) * jnp.float32(1.0 / _BW)
            idx = jnp.clip(t.astype(jnp.int32), 0, _NUM_BUCKETS - 1)
            delta = (
                xc - jnp.float32(_LOWER) + jnp.float32(_BW)
                - idx.astype(jnp.float32) * jnp.float32(_BW)
            )
            g1 = plsc.load_gather(t1_v, [idx])
            g2 = plsc.load_gather(t2_v, [idx])
            z = g1 + delta * g2
            out_v[pl.ds(off, 16)] = (
                jnp.float32(1.0) / (jnp.float32(1.0) + jnp.exp(-z))
            )

        pltpu.sync_copy(out_v, out_hbm.at[pl.ds(base, _BATCH)])


def kernel(x, weights, bias):
    wp = jnp.pad(weights, ((0, 0), (0, _KPAD - _NUM_BUCKETS))).reshape(-1)
    bp = jnp.pad(bias, (0, 32 - _UNITS))
    xt = x.T.reshape(-1)
    mesh = plsc.VectorSubcoreMesh(core_axis_name="c", subcore_axis_name="s")
    run = functools.partial(
        pl.kernel,
        mesh=mesh,
        out_type=jax.ShapeDtypeStruct((_UNITS * _BATCH,), jnp.float32),
        scratch_types=[
            pltpu.VMEM((_BATCH,), jnp.float32),
            pltpu.VMEM((_BATCH,), jnp.float32),
            pltpu.VMEM((_KPAD,), jnp.float32),
            pltpu.VMEM((32,), jnp.float32),
            pltpu.VMEM((_KPAD,), jnp.float32),
            pltpu.VMEM((_KPAD,), jnp.float32),
            pltpu.SemaphoreType.DMA,
        ],
        compiler_params=pltpu.CompilerParams(needs_layout_passes=False),
    )(_sc_body)
    out = run(xt, wp, bp)
    return out.reshape(_UNITS, _BATCH).T
